# baseline (speedup 1.0000x reference)
import functools
import math

import jax
import jax.numpy as jnp
from jax import lax
from jax.experimental import pallas as pl
from jax.experimental.pallas import tpu as pltpu
from jax.experimental.pallas import tpu_sc as plsc

_RANK = 32
_BATCH = 4096
_K = 70
_E = _BATCH * _K
_MIN_NORM = 1e-15
_T_LVL0 = math.tanh(0.5)
_T_LVL1 = math.tanh(1.0)
_T_LEAF = math.tanh(2.0)

_NW = 32
_CHUNK = 128
_ROWS_W = _BATCH // _NW
_E_W = _E // _NW
_NCH = _E_W // _CHUNK
_WIN = 10
_NWIN = _NCH // _WIN


def _sc_dots(ue2, ie2, ne2, nsup, usup, isup0, isup1, urow):
    mesh = plsc.VectorSubcoreMesh(core_axis_name="c", subcore_axis_name="s")
    flat = jax.ShapeDtypeStruct((_E,), jnp.float32)

    @functools.partial(
        pl.kernel,
        out_type=(flat, flat, flat, flat, flat,
                  jax.ShapeDtypeStruct((_NW, 8, 128), jnp.float32)),
        mesh=mesh,
        compiler_params=pltpu.CompilerParams(
            needs_layout_passes=False, use_tc_tiling_on_sc=False),
        scratch_types=[
            pltpu.VMEM((_NCH, _CHUNK), jnp.int32),
            pltpu.VMEM((1, 128), jnp.int32),
            pltpu.VMEM((1, 128), jnp.int32),
            pltpu.VMEM((1, 128), jnp.int32),
            pltpu.VMEM((_E_W,), jnp.int32),
            pltpu.VMEM((_ROWS_W, _RANK), jnp.float32),
            pltpu.VMEM((_ROWS_W, _RANK), jnp.float32),
            pltpu.VMEM((_ROWS_W, _RANK), jnp.float32),
            pltpu.VMEM((2, _CHUNK, _RANK), jnp.float32),
            pltpu.VMEM((128,), jnp.float32),
            pltpu.VMEM((128,), jnp.float32),
            pltpu.VMEM((128,), jnp.float32),
            pltpu.VMEM((128,), jnp.float32),
            pltpu.VMEM((128,), jnp.float32),
            pltpu.VMEM((_WIN * _CHUNK,), jnp.float32),
            pltpu.VMEM((_WIN * _CHUNK,), jnp.float32),
            pltpu.VMEM((_WIN * _CHUNK,), jnp.float32),
            pltpu.VMEM((_WIN * _CHUNK,), jnp.float32),
            pltpu.VMEM((_WIN * _CHUNK,), jnp.float32),
            pltpu.SemaphoreType.DMA,
            pltpu.SemaphoreType.DMA,
            pltpu.SemaphoreType.DMA,
        ],
    )
    def k(ue_r, ie_r, ne_r, nsup_r, usup_r, isup0_r, isup1_r, urow_r,
          du_o, di_o, nn_o, uue_o, iie_o, rs_o,
          niv, uiv, iiv0, iiv1, urowv,
          ubuf, ibuf0, ibuf1, nbuf, uu_b, ii0_b, ii1_b, ui0_b, ui1_b,
          odu, odi, onn, ouu, oii, s0, s1, s2):
        wid = lax.axis_index("s") * 2 + lax.axis_index("c")
        ebase = wid * _E_W

        pltpu.sync_copy(nsup_r.at[wid], niv)
        pltpu.sync_copy(usup_r.at[wid], uiv)
        pltpu.sync_copy(isup0_r.at[wid], iiv0)
        pltpu.sync_copy(isup1_r.at[wid], iiv1)
        pltpu.sync_copy(urow_r.at[pl.ds(ebase, _E_W)], urowv)

        pltpu.async_copy(ue_r.at[uiv.at[0]], ubuf, s2)
        pltpu.async_copy(ie_r.at[iiv0.at[0]], ibuf0, s2)
        pltpu.async_copy(ie_r.at[iiv1.at[0]], ibuf1, s2)

        def nstart(tg, buf, sem):
            pltpu.async_copy(ne_r.at[niv.at[tg]], nbuf.at[buf], sem)

        def nwait(buf, sem):
            pltpu.make_async_copy(ne_r.at[niv.at[0]], nbuf.at[buf], sem).wait()

        nstart(0, 0, s0)
        nstart(1, 1, s1)

        for _ in range(3):
            pltpu.make_async_copy(ie_r.at[iiv0.at[0]], ibuf0, s2).wait()

        @pl.loop(0, _ROWS_W // 16)
        def _(g):
            rv = lax.iota(jnp.int32, 16) + g * 16
            z = jnp.zeros((16,), jnp.float32)
            uu = ii0 = ii1 = ui0 = ui1 = z
            for d in range(_RANK):
                dv = jnp.full((16,), d, jnp.int32)
                uf = plsc.load_gather(ubuf, [rv, dv])
                f0 = plsc.load_gather(ibuf0, [rv, dv])
                f1 = plsc.load_gather(ibuf1, [rv, dv])
                uu = uu + uf * uf
                ii0 = ii0 + f0 * f0
                ii1 = ii1 + f1 * f1
                ui0 = ui0 + uf * f0
                ui1 = ui1 + uf * f1
            sl = pl.ds(g * 16, 16)
            uu_b[sl] = uu
            ii0_b[sl] = ii0
            ii1_b[sl] = ii1
            ui0_b[sl] = ui0
            ui1_b[sl] = ui1

        pltpu.sync_copy(uu_b, rs_o.at[wid, 0])
        pltpu.sync_copy(ii0_b, rs_o.at[wid, 1])
        pltpu.sync_copy(ii1_b, rs_o.at[wid, 2])
        pltpu.sync_copy(ui0_b, rs_o.at[wid, 3])
        pltpu.sync_copy(ui1_b, rs_o.at[wid, 4])

        def compute_chunk(buf, tg, woff):
            @pl.loop(0, _CHUNK // 16)
            def _(g):
                go = tg * _CHUNK + g * 16
                wo = woff * _CHUNK + g * 16
                urv = urowv[pl.ds(go, 16)]
                nrow = lax.iota(jnp.int32, 16) + g * 16
                z = jnp.zeros((16,), jnp.float32)
                a_nn = a_du = a_di = z
                for d in range(_RANK):
                    dv = jnp.full((16,), d, jnp.int32)
                    nf = plsc.load_gather(nbuf.at[buf], [nrow, dv])
                    uf = plsc.load_gather(ubuf, [urv, dv])
                    f0 = plsc.load_gather(ibuf0, [urv, dv])
                    a_nn = a_nn + nf * nf
                    a_du = a_du + nf * uf
                    a_di = a_di + nf * f0
                sl = pl.ds(wo, 16)
                odu[sl] = a_du
                odi[sl] = a_di
                onn[sl] = a_nn
                ouu[sl] = plsc.load_gather(uu_b, [urv])
                oii[sl] = plsc.load_gather(ii0_b, [urv])

        @pl.loop(0, _NWIN)
        def _(tt):
            @pl.loop(0, _WIN, step=2)
            def _(cc):
                t0 = tt * _WIN + cc
                nwait(0, s0)
                compute_chunk(0, t0, cc)

                @pl.when(t0 + 2 < _NCH)
                def _():
                    nstart(t0 + 2, 0, s0)

                nwait(1, s1)
                compute_chunk(1, t0 + 1, cc + 1)

                @pl.when(t0 + 3 < _NCH)
                def _():
                    nstart(t0 + 3, 1, s1)

            wsl = pl.ds(ebase + tt * _WIN * _CHUNK, _WIN * _CHUNK)
            pltpu.sync_copy(odu, du_o.at[wsl])
            pltpu.sync_copy(odi, di_o.at[wsl])
            pltpu.sync_copy(onn, nn_o.at[wsl])
            pltpu.sync_copy(ouu, uue_o.at[wsl])
            pltpu.sync_copy(oii, iie_o.at[wsl])

    return k(ue2, ie2, ne2, nsup, usup, isup0, isup1, urow)


_TBLK = 4096


def _tr_body(i_ref, o_ref):
    t = i_ref[...].T
    t3 = t.reshape(_TBLK // 4, 4, _RANK)
    for j in range(4):
        o_ref[:, j * _RANK:(j + 1) * _RANK] = t3[:, j, :]


def _tc_transpose(xt):
    n = xt.shape[1]
    grid = ((n + _TBLK - 1) // _TBLK,)
    return pl.pallas_call(
        _tr_body,
        grid=grid,
        in_specs=[pl.BlockSpec((_RANK, _TBLK), lambda i: (0, i))],
        out_specs=pl.BlockSpec((_TBLK // 4, 128), lambda i: (i, 0)),
        out_shape=jax.ShapeDtypeStruct(((n + 3) // 4, 128), jnp.float32),
    )(xt)


def _tr_rows_body(i_ref, o_ref):
    o_ref[...] = i_ref[...].T


def _tc_transpose_rows(xt):
    n = xt.shape[1]
    grid = ((n + _TBLK - 1) // _TBLK,)
    return pl.pallas_call(
        _tr_rows_body,
        grid=grid,
        in_specs=[pl.BlockSpec((_RANK, _TBLK), lambda i: (0, i))],
        out_specs=pl.BlockSpec((_TBLK, _RANK), lambda i: (i, 0)),
        out_shape=jax.ShapeDtypeStruct((n, _RANK), jnp.float32),
    )(xt)


def _softplus(x):
    return jnp.maximum(x, 0.0) + jnp.log(1.0 + jnp.exp(-jnp.abs(x)))


def _artanh(x):
    x = jnp.clip(x, -1.0 + 1e-7, 1.0 - 1e-7)
    return 0.5 * jnp.log((1.0 + x) / (1.0 - x))


def _dist_scalar(x2, y2, xy, c, sqrt_c):
    a = 1.0 - 2.0 * c * xy + c * y2
    b = 1.0 - c * x2
    num2 = a * a * x2 - 2.0 * a * b * xy + b * b * y2
    den = 1.0 - 2.0 * c * xy + (c * c) * x2 * y2
    pn = jnp.sqrt(jnp.maximum(num2, 0.0)) / jnp.maximum(den, _MIN_NORM)
    return 2.0 / sqrt_c * _artanh(sqrt_c * pn)


_EB = 320


def _tc_entry_body(c_ref, du_ref, di_ref, nn_ref, uu_ref, ii_ref, rs_ref,
                   dun_ref, din_ref):
    c = _softplus(c_ref[0, 0])
    sqrt_c = jnp.sqrt(c)
    du, di, nn = du_ref[...], di_ref[...], nn_ref[...]
    uu, ii, rsel = uu_ref[...], ii_ref[...], rs_ref[...]
    tr = rsel * _T_LVL1 + (1.0 - rsel) * _T_LVL0
    su = _T_LEAF / jnp.maximum(jnp.sqrt(uu), _MIN_NORM)
    si = _T_LEAF / jnp.maximum(jnp.sqrt(ii), _MIN_NORM)
    sn = tr / jnp.maximum(jnp.sqrt(nn), _MIN_NORM)
    y2 = sn * sn * nn
    dun_ref[...] = _dist_scalar(su * su * uu, y2, su * sn * du, c, sqrt_c)
    din_ref[...] = _dist_scalar(si * si * ii, y2, si * sn * di, c, sqrt_c)


def _tc_entry(c2, du, di, nn, uu, ii, rsel, interpret=False):
    n_blk = _E // 128 // _EB
    bs = lambda: pl.BlockSpec((_EB, 128), lambda i: (i, 0))
    return pl.pallas_call(
        _tc_entry_body,
        grid=(n_blk,),
        in_specs=[
            pl.BlockSpec((1, 1), lambda i: (0, 0), memory_space=pltpu.SMEM),
            bs(), bs(), bs(), bs(), bs(), bs(),
        ],
        out_specs=[bs(), bs()],
        out_shape=(
            jax.ShapeDtypeStruct((_E // 128, 128), jnp.float32),
            jax.ShapeDtypeStruct((_E // 128, 128), jnp.float32),
        ),
        interpret=interpret,
    )(c2, du, di, nn, uu, ii, rsel)


def _tc_uid_body(c_ref, rs_ref, out_ref):
    c = _softplus(c_ref[0, 0])
    sqrt_c = jnp.sqrt(c)
    rs = rs_ref[...]
    uu, ii0, ii1 = rs[:, 0, :], rs[:, 1, :], rs[:, 2, :]
    ui0, ui1 = rs[:, 3, :], rs[:, 4, :]
    su = _T_LEAF / jnp.maximum(jnp.sqrt(uu), _MIN_NORM)
    si0 = _T_LEAF / jnp.maximum(jnp.sqrt(ii0), _MIN_NORM)
    si1 = _T_LEAF / jnp.maximum(jnp.sqrt(ii1), _MIN_NORM)
    x2 = su * su * uu
    out_ref[:, 0, :] = _dist_scalar(x2, si0 * si0 * ii0, su * si0 * ui0,
                                    c, sqrt_c)
    out_ref[:, 1, :] = _dist_scalar(x2, si1 * si1 * ii1, su * si1 * ui1,
                                    c, sqrt_c)


def _tc_uid(c2, rs, interpret=False):
    return pl.pallas_call(
        _tc_uid_body,
        grid=(1,),
        in_specs=[
            pl.BlockSpec((1, 1), lambda i: (0, 0), memory_space=pltpu.SMEM),
            pl.BlockSpec((_NW, 8, 128), lambda i: (0, 0, 0)),
        ],
        out_specs=pl.BlockSpec((_NW, 2, 128), lambda i: (0, 0, 0)),
        out_shape=jax.ShapeDtypeStruct((_NW, 2, 128), jnp.float32),
        interpret=interpret,
    )(c2, rs)


def kernel(input_tensor, nodes_ind, user_embeddings, item_embeddings,
           node_embeddings, c_var):
    it32 = input_tensor.astype(jnp.int32)
    nid = nodes_ind.astype(jnp.int32).reshape(-1)

    nsup = nid.reshape(_NW, _NCH, _CHUNK)
    e = jnp.arange(_E, dtype=jnp.int32)
    urow = (e // _K) % _ROWS_W

    usup = it32[:, 0].reshape(_NW, 1, 128)
    isup0 = it32[:, 1].reshape(_NW, 1, 128)
    isup1 = it32[:, 2].reshape(_NW, 1, 128)

    du, di, nn, uu_e, ii_e, rs = _sc_dots(
        user_embeddings, item_embeddings, node_embeddings,
        nsup, usup, isup0, isup1, urow)

    rsel = ((e % _K) >= 20).astype(jnp.float32).reshape(_E // 128, 128)
    c2 = c_var.reshape(1, 1)
    dun, din = _tc_entry(c2, du.reshape(_E // 128, 128),
                         di.reshape(_E // 128, 128),
                         nn.reshape(_E // 128, 128),
                         uu_e.reshape(_E // 128, 128),
                         ii_e.reshape(_E // 128, 128), rsel)

    uid2 = _tc_uid(c2, rs)

    und = dun.reshape(_BATCH, _K)
    pind = din.reshape(_BATCH, _K)
    uid = uid2.transpose(0, 2, 1).reshape(_BATCH, 2)
    return (und, pind, uid)


# speedup vs baseline: 1.1832x; 1.1832x over previous
import functools
import math

import jax
import jax.numpy as jnp
from jax import lax
from jax.experimental import pallas as pl
from jax.experimental.pallas import tpu as pltpu
from jax.experimental.pallas import tpu_sc as plsc

_RANK = 32
_BATCH = 4096
_K = 70
_E = _BATCH * _K
_MIN_NORM = 1e-15
_T_LVL0 = math.tanh(0.5)
_T_LVL1 = math.tanh(1.0)
_T_LEAF = math.tanh(2.0)

_NW = 32
_CHUNK = 128
_ROWS_W = _BATCH // _NW
_E_W = _E // _NW
_NCH = _E_W // _CHUNK
_WIN = 10
_NWIN = _NCH // _WIN


def _sc_dots(ue2, ie2, ne2, nsup, usup, isup0, isup1, urow):
    mesh = plsc.VectorSubcoreMesh(core_axis_name="c", subcore_axis_name="s")
    flat = jax.ShapeDtypeStruct((_E,), jnp.float32)

    @functools.partial(
        pl.kernel,
        out_type=(flat, flat, flat, flat, flat,
                  jax.ShapeDtypeStruct((_NW, 8, 128), jnp.float32)),
        mesh=mesh,
        compiler_params=pltpu.CompilerParams(
            needs_layout_passes=False, use_tc_tiling_on_sc=False),
        scratch_types=[
            pltpu.VMEM((_NCH, _CHUNK), jnp.int32),
            pltpu.VMEM((1, 128), jnp.int32),
            pltpu.VMEM((1, 128), jnp.int32),
            pltpu.VMEM((1, 128), jnp.int32),
            pltpu.VMEM((_E_W,), jnp.int32),
            pltpu.VMEM((_ROWS_W, _RANK), jnp.float32),
            pltpu.VMEM((_ROWS_W, _RANK), jnp.float32),
            pltpu.VMEM((_ROWS_W, _RANK), jnp.float32),
            pltpu.VMEM((2, _CHUNK, _RANK), jnp.float32),
            pltpu.VMEM((128,), jnp.float32),
            pltpu.VMEM((128,), jnp.float32),
            pltpu.VMEM((128,), jnp.float32),
            pltpu.VMEM((128,), jnp.float32),
            pltpu.VMEM((128,), jnp.float32),
            pltpu.VMEM((_WIN * _CHUNK,), jnp.float32),
            pltpu.VMEM((_WIN * _CHUNK,), jnp.float32),
            pltpu.VMEM((_WIN * _CHUNK,), jnp.float32),
            pltpu.VMEM((_WIN * _CHUNK,), jnp.float32),
            pltpu.VMEM((_WIN * _CHUNK,), jnp.float32),
            pltpu.SemaphoreType.DMA,
            pltpu.SemaphoreType.DMA,
            pltpu.SemaphoreType.DMA,
        ],
    )
    def k(ue_r, ie_r, ne_r, nsup_r, usup_r, isup0_r, isup1_r, urow_r,
          du_o, di_o, nn_o, uue_o, iie_o, rs_o,
          niv, uiv, iiv0, iiv1, urowv,
          ubuf, ibuf0, ibuf1, nbuf, uu_b, ii0_b, ii1_b, ui0_b, ui1_b,
          odu, odi, onn, ouu, oii, s0, s1, s2):
        wid = lax.axis_index("s") * 2 + lax.axis_index("c")
        ebase = wid * _E_W

        pltpu.sync_copy(nsup_r.at[wid], niv)
        pltpu.sync_copy(usup_r.at[wid], uiv)
        pltpu.sync_copy(isup0_r.at[wid], iiv0)
        pltpu.sync_copy(isup1_r.at[wid], iiv1)
        pltpu.sync_copy(urow_r.at[pl.ds(ebase, _E_W)], urowv)

        pltpu.async_copy(ue_r.at[uiv.at[0]], ubuf, s2)
        pltpu.async_copy(ie_r.at[iiv0.at[0]], ibuf0, s2)
        pltpu.async_copy(ie_r.at[iiv1.at[0]], ibuf1, s2)

        def nstart(tg, buf, sem):
            pltpu.async_copy(ne_r.at[niv.at[tg]], nbuf.at[buf], sem)

        def nwait(buf, sem):
            pltpu.make_async_copy(ne_r.at[niv.at[0]], nbuf.at[buf], sem).wait()

        nstart(0, 0, s0)
        nstart(1, 1, s1)

        for _ in range(3):
            pltpu.make_async_copy(ie_r.at[iiv0.at[0]], ibuf0, s2).wait()

        @pl.loop(0, _ROWS_W // 16)
        def _(g):
            rv = lax.iota(jnp.int32, 16) + g * 16
            z = jnp.zeros((16,), jnp.float32)
            lane = lax.iota(jnp.int32, 16)
            uu = ii0 = ii1 = ui0 = ui1 = z
            for d in range(_RANK):
                dv = (lane + d) & (_RANK - 1)
                uf = plsc.load_gather(ubuf, [rv, dv])
                f0 = plsc.load_gather(ibuf0, [rv, dv])
                f1 = plsc.load_gather(ibuf1, [rv, dv])
                uu = uu + uf * uf
                ii0 = ii0 + f0 * f0
                ii1 = ii1 + f1 * f1
                ui0 = ui0 + uf * f0
                ui1 = ui1 + uf * f1
            sl = pl.ds(g * 16, 16)
            uu_b[sl] = uu
            ii0_b[sl] = ii0
            ii1_b[sl] = ii1
            ui0_b[sl] = ui0
            ui1_b[sl] = ui1

        pltpu.sync_copy(uu_b, rs_o.at[wid, 0])
        pltpu.sync_copy(ii0_b, rs_o.at[wid, 1])
        pltpu.sync_copy(ii1_b, rs_o.at[wid, 2])
        pltpu.sync_copy(ui0_b, rs_o.at[wid, 3])
        pltpu.sync_copy(ui1_b, rs_o.at[wid, 4])

        def compute_chunk(buf, tg, woff):
            @pl.loop(0, _CHUNK // 16)
            def _(g):
                go = tg * _CHUNK + g * 16
                wo = woff * _CHUNK + g * 16
                urv = urowv[pl.ds(go, 16)]
                lane = lax.iota(jnp.int32, 16)
                nrow = lane + g * 16
                z = jnp.zeros((16,), jnp.float32)
                a_nn = a_du = a_di = z
                for d in range(_RANK):
                    dv = (lane + d) & (_RANK - 1)
                    nf = plsc.load_gather(nbuf.at[buf], [nrow, dv])
                    uf = plsc.load_gather(ubuf, [urv, dv])
                    f0 = plsc.load_gather(ibuf0, [urv, dv])
                    a_nn = a_nn + nf * nf
                    a_du = a_du + nf * uf
                    a_di = a_di + nf * f0
                sl = pl.ds(wo, 16)
                odu[sl] = a_du
                odi[sl] = a_di
                onn[sl] = a_nn
                ouu[sl] = plsc.load_gather(uu_b, [urv])
                oii[sl] = plsc.load_gather(ii0_b, [urv])

        @pl.loop(0, _NWIN)
        def _(tt):
            @pl.loop(0, _WIN, step=2)
            def _(cc):
                t0 = tt * _WIN + cc
                nwait(0, s0)
                compute_chunk(0, t0, cc)

                @pl.when(t0 + 2 < _NCH)
                def _():
                    nstart(t0 + 2, 0, s0)

                nwait(1, s1)
                compute_chunk(1, t0 + 1, cc + 1)

                @pl.when(t0 + 3 < _NCH)
                def _():
                    nstart(t0 + 3, 1, s1)

            wsl = pl.ds(ebase + tt * _WIN * _CHUNK, _WIN * _CHUNK)
            pltpu.sync_copy(odu, du_o.at[wsl])
            pltpu.sync_copy(odi, di_o.at[wsl])
            pltpu.sync_copy(onn, nn_o.at[wsl])
            pltpu.sync_copy(ouu, uue_o.at[wsl])
            pltpu.sync_copy(oii, iie_o.at[wsl])

    return k(ue2, ie2, ne2, nsup, usup, isup0, isup1, urow)


_TBLK = 4096


def _tr_body(i_ref, o_ref):
    t = i_ref[...].T
    t3 = t.reshape(_TBLK // 4, 4, _RANK)
    for j in range(4):
        o_ref[:, j * _RANK:(j + 1) * _RANK] = t3[:, j, :]


def _tc_transpose(xt):
    n = xt.shape[1]
    grid = ((n + _TBLK - 1) // _TBLK,)
    return pl.pallas_call(
        _tr_body,
        grid=grid,
        in_specs=[pl.BlockSpec((_RANK, _TBLK), lambda i: (0, i))],
        out_specs=pl.BlockSpec((_TBLK // 4, 128), lambda i: (i, 0)),
        out_shape=jax.ShapeDtypeStruct(((n + 3) // 4, 128), jnp.float32),
    )(xt)


def _tr_rows_body(i_ref, o_ref):
    o_ref[...] = i_ref[...].T


def _tc_transpose_rows(xt):
    n = xt.shape[1]
    grid = ((n + _TBLK - 1) // _TBLK,)
    return pl.pallas_call(
        _tr_rows_body,
        grid=grid,
        in_specs=[pl.BlockSpec((_RANK, _TBLK), lambda i: (0, i))],
        out_specs=pl.BlockSpec((_TBLK, _RANK), lambda i: (i, 0)),
        out_shape=jax.ShapeDtypeStruct((n, _RANK), jnp.float32),
    )(xt)


def _softplus(x):
    return jnp.maximum(x, 0.0) + jnp.log(1.0 + jnp.exp(-jnp.abs(x)))


def _artanh(x):
    x = jnp.clip(x, -1.0 + 1e-7, 1.0 - 1e-7)
    return 0.5 * jnp.log((1.0 + x) / (1.0 - x))


def _dist_scalar(x2, y2, xy, c, sqrt_c):
    a = 1.0 - 2.0 * c * xy + c * y2
    b = 1.0 - c * x2
    num2 = a * a * x2 - 2.0 * a * b * xy + b * b * y2
    den = 1.0 - 2.0 * c * xy + (c * c) * x2 * y2
    pn = jnp.sqrt(jnp.maximum(num2, 0.0)) / jnp.maximum(den, _MIN_NORM)
    return 2.0 / sqrt_c * _artanh(sqrt_c * pn)


_EB = 320


def _tc_entry_body(c_ref, du_ref, di_ref, nn_ref, uu_ref, ii_ref, rs_ref,
                   dun_ref, din_ref):
    c = _softplus(c_ref[0, 0])
    sqrt_c = jnp.sqrt(c)
    du, di, nn = du_ref[...], di_ref[...], nn_ref[...]
    uu, ii, rsel = uu_ref[...], ii_ref[...], rs_ref[...]
    tr = rsel * _T_LVL1 + (1.0 - rsel) * _T_LVL0
    su = _T_LEAF / jnp.maximum(jnp.sqrt(uu), _MIN_NORM)
    si = _T_LEAF / jnp.maximum(jnp.sqrt(ii), _MIN_NORM)
    sn = tr / jnp.maximum(jnp.sqrt(nn), _MIN_NORM)
    y2 = sn * sn * nn
    dun_ref[...] = _dist_scalar(su * su * uu, y2, su * sn * du, c, sqrt_c)
    din_ref[...] = _dist_scalar(si * si * ii, y2, si * sn * di, c, sqrt_c)


def _tc_entry(c2, du, di, nn, uu, ii, rsel, interpret=False):
    n_blk = _E // 128 // _EB
    bs = lambda: pl.BlockSpec((_EB, 128), lambda i: (i, 0))
    return pl.pallas_call(
        _tc_entry_body,
        grid=(n_blk,),
        in_specs=[
            pl.BlockSpec((1, 1), lambda i: (0, 0), memory_space=pltpu.SMEM),
            bs(), bs(), bs(), bs(), bs(), bs(),
        ],
        out_specs=[bs(), bs()],
        out_shape=(
            jax.ShapeDtypeStruct((_E // 128, 128), jnp.float32),
            jax.ShapeDtypeStruct((_E // 128, 128), jnp.float32),
        ),
        interpret=interpret,
    )(c2, du, di, nn, uu, ii, rsel)


def _tc_uid_body(c_ref, rs_ref, out_ref):
    c = _softplus(c_ref[0, 0])
    sqrt_c = jnp.sqrt(c)
    rs = rs_ref[...]
    uu, ii0, ii1 = rs[:, 0, :], rs[:, 1, :], rs[:, 2, :]
    ui0, ui1 = rs[:, 3, :], rs[:, 4, :]
    su = _T_LEAF / jnp.maximum(jnp.sqrt(uu), _MIN_NORM)
    si0 = _T_LEAF / jnp.maximum(jnp.sqrt(ii0), _MIN_NORM)
    si1 = _T_LEAF / jnp.maximum(jnp.sqrt(ii1), _MIN_NORM)
    x2 = su * su * uu
    out_ref[:, 0, :] = _dist_scalar(x2, si0 * si0 * ii0, su * si0 * ui0,
                                    c, sqrt_c)
    out_ref[:, 1, :] = _dist_scalar(x2, si1 * si1 * ii1, su * si1 * ui1,
                                    c, sqrt_c)


def _tc_uid(c2, rs, interpret=False):
    return pl.pallas_call(
        _tc_uid_body,
        grid=(1,),
        in_specs=[
            pl.BlockSpec((1, 1), lambda i: (0, 0), memory_space=pltpu.SMEM),
            pl.BlockSpec((_NW, 8, 128), lambda i: (0, 0, 0)),
        ],
        out_specs=pl.BlockSpec((_NW, 2, 128), lambda i: (0, 0, 0)),
        out_shape=jax.ShapeDtypeStruct((_NW, 2, 128), jnp.float32),
        interpret=interpret,
    )(c2, rs)


def kernel(input_tensor, nodes_ind, user_embeddings, item_embeddings,
           node_embeddings, c_var):
    it32 = input_tensor.astype(jnp.int32)
    nid = nodes_ind.astype(jnp.int32).reshape(-1)

    nsup = nid.reshape(_NW, _NCH, _CHUNK)
    e = jnp.arange(_E, dtype=jnp.int32)
    urow = (e // _K) % _ROWS_W

    usup = it32[:, 0].reshape(_NW, 1, 128)
    isup0 = it32[:, 1].reshape(_NW, 1, 128)
    isup1 = it32[:, 2].reshape(_NW, 1, 128)

    du, di, nn, uu_e, ii_e, rs = _sc_dots(
        user_embeddings, item_embeddings, node_embeddings,
        nsup, usup, isup0, isup1, urow)

    rsel = ((e % _K) >= 20).astype(jnp.float32).reshape(_E // 128, 128)
    c2 = c_var.reshape(1, 1)
    dun, din = _tc_entry(c2, du.reshape(_E // 128, 128),
                         di.reshape(_E // 128, 128),
                         nn.reshape(_E // 128, 128),
                         uu_e.reshape(_E // 128, 128),
                         ii_e.reshape(_E // 128, 128), rsel)

    uid2 = _tc_uid(c2, rs)

    und = dun.reshape(_BATCH, _K)
    pind = din.reshape(_BATCH, _K)
    uid = uid2.transpose(0, 2, 1).reshape(_BATCH, 2)
    return (und, pind, uid)
